# baseline (speedup 1.0000x reference)
import functools

import jax
import jax.numpy as jnp
from jax import lax
from jax.experimental import pallas as pl
from jax.experimental.pallas import tpu as pltpu
from jax.experimental.pallas import tpu_sc as plsc

N = 100000
E = 3200000
HIDDEN = 128
OUT = 128
G = 64

NC = 2
NS = 16
NW = NC * NS

CH = 5000
CHUNKS_PER_TILE = E // (NW * CH)
SLICE = 6400
NP_SC = NS * SLICE


def _sc_agg(xa, sa, da, xp, sp, dp, xn, sn, dn, zeros, out, agg_sh, x_sh,
            tailbuf, src0, dst0, vals0, src1, dst1, vals1, src2, dst2, vals2,
            src3, dst3, vals3, ig0, ig1, ig2, ig3, sg0, sg1, sg2, sg3,
            ss0, ss1, ss2, ss3):
    cid = lax.axis_index("c")
    sid = lax.axis_index("s")
    wid = sid * NC + cid
    nbase = sid * SLICE
    cbase = wid * CHUNKS_PER_TILE

    srcs = (src0, src1, src2, src3)
    dsts = (dst0, dst1, dst2, dst3)
    vlss = (vals0, vals1, vals2, vals3)
    igs = (ig0, ig1, ig2, ig3)
    sgs = (sg0, sg1, sg2, sg3)
    sss = (ss0, ss1, ss2, ss3)
    NB = 4

    for g, (x_hbm, s_hbm, d_hbm) in enumerate(
            ((xa, sa, da), (xp, sp, dp), (xn, sn, dn))):
        pltpu.sync_copy(zeros.at[pl.ds(nbase, SLICE)],
                        agg_sh.at[pl.ds(nbase, SLICE)])
        XTAIL = N - 15 * SLICE
        @pl.when(sid < 15)
        def _sx():
            pltpu.sync_copy(x_hbm.at[pl.ds(nbase, SLICE)],
                            x_sh.at[pl.ds(nbase, SLICE)])
        @pl.when(sid == 15)
        def _sxt():
            pltpu.sync_copy(x_hbm.at[pl.ds(15 * SLICE, XTAIL)], tailbuf)
            pltpu.sync_copy(tailbuf, x_sh.at[pl.ds(15 * SLICE, XTAIL)])
        plsc.subcore_barrier()

        def start_idx(k):
            p = k % NB
            base = (cbase + k) * CH
            pltpu.async_copy(s_hbm.at[pl.ds(base, CH)], srcs[p], igs[p])
            pltpu.async_copy(d_hbm.at[pl.ds(base, CH)], dsts[p], igs[p])

        def wait_idx(p):
            pltpu.make_async_copy(s_hbm.at[pl.ds(0, CH)], srcs[p], igs[p]).wait()
            pltpu.make_async_copy(d_hbm.at[pl.ds(0, CH)], dsts[p], igs[p]).wait()

        def start_gather(p):
            pltpu.async_copy(x_sh.at[srcs[p]], vlss[p], sgs[p])

        def wait_gather(p):
            pltpu.make_async_copy(x_sh.at[srcs[p]], vlss[p], sgs[p]).wait()

        def start_scatter(p):
            pltpu.async_copy(vlss[p], agg_sh.at[dsts[p]], sss[p], add=True)

        def wait_scatter(p):
            pltpu.make_async_copy(vlss[p], agg_sh.at[dsts[p]], sss[p]).wait()

        CPT = CHUNKS_PER_TILE
        start_idx(0)
        for k in range(CPT):
            p = k % NB
            wait_idx(p)
            if k >= 3:
                wait_scatter((k - 3) % NB)
            if k + 1 < CPT:
                start_idx(k + 1)
            start_gather(p)
            if k >= 2:
                q = (k - 2) % NB
                wait_gather(q)
                start_scatter(q)
        for k in (CPT - 2, CPT - 1):
            wait_gather(k % NB)
            start_scatter(k % NB)
        for k in (CPT - 3, CPT - 2, CPT - 1):
            wait_scatter(k % NB)
        plsc.subcore_barrier()

        obase = (g * NC + cid) * NP_SC + nbase
        pltpu.sync_copy(agg_sh.at[pl.ds(nbase, SLICE)],
                        out.at[pl.ds(obase, SLICE)])
        plsc.subcore_barrier()


def _sc_call(xa, ea, xp, ep, xn, en):
    mesh = plsc.VectorSubcoreMesh(core_axis_name="c", subcore_axis_name="s",
                                  num_cores=NC, num_subcores=NS)
    zeros = jnp.zeros((NP_SC,), jnp.float32)
    return pl.kernel(
        _sc_agg,
        out_type=jax.ShapeDtypeStruct((3 * NC * NP_SC,), jnp.float32),
        mesh=mesh,
        scratch_types=[
            pltpu.VMEM_SHARED((NP_SC,), jnp.float32),
            pltpu.VMEM_SHARED((NP_SC,), jnp.float32),
            pltpu.VMEM((N - 15 * SLICE,), jnp.float32),
        ] + [pltpu.VMEM((CH,), jnp.int32),
             pltpu.VMEM((CH,), jnp.int32),
             pltpu.VMEM((CH,), jnp.float32)] * 4
          + [pltpu.SemaphoreType.DMA] * 12,
    )(xa, ea[0], ea[1], xp, ep[0], ep[1], xn, en[0], en[1], zeros)


BN = 4096
NBLK = NP_SC // BN


def _tc_body(xa, xp, xn, ba, bp, bn, a0, a1, a2, a3, a4, a5,
             w1, b1, w2t, b2, fwt, fb, out_ref, acc, cnt):
    b = pl.program_id(0)

    @pl.when(b == 0)
    def _init():
        acc[...] = jnp.zeros_like(acc)
        cnt[...] = jnp.zeros_like(cnt)

    lane = lax.broadcasted_iota(jnp.int32, (1, BN), 1)
    valid = (b * BN + lane) < N
    gid = lax.broadcasted_iota(jnp.int32, (G, BN), 0)
    nt = (((1,), (1,)), ((), ()))

    for g, (xr, btr, p0, p1) in enumerate(
            ((xa, ba, a0, a1), (xp, bp, a2, a3), (xn, bn, a4, a5))):
        h = (xr[...] + p0[...] + p1[...]).reshape(1, BN)
        rt = jnp.maximum(w1[...] * h + b1[...], 0.0)
        rt = jnp.where(valid, rt, 0.0)
        bt = btr[...].reshape(1, BN)
        oh = ((gid == bt) & valid).astype(jnp.float32)
        acc[:, G * g:G * (g + 1)] += lax.dot_general(
            rt, oh, nt, preferred_element_type=jnp.float32)
        cnt[:, g:g + 1] += jnp.sum(oh, axis=1, keepdims=True)

    @pl.when(b == NBLK - 1)
    def _fin():
        for g in range(3):
            t = jnp.dot(w2t[...], acc[:, G * g:G * (g + 1)],
                        preferred_element_type=jnp.float32)
            t = t + lax.dot_general(b2[...], cnt[:, g:g + 1], nt,
                                    preferred_element_type=jnp.float32)
            o = jnp.dot(fwt[...], t, preferred_element_type=jnp.float32)
            out_ref[g] = o + fb[...]


def _tc_call(xa, xp, xn, ba, bp, bn, agg, W1, b1, W2, b2, fc_W, fc_b):
    node_spec = pl.BlockSpec((BN,), lambda b: (b,))
    agg_specs = [
        pl.BlockSpec((BN,), lambda b, r=r: (r * (NP_SC // BN) + b,))
        for r in range(6)
    ]
    return pl.pallas_call(
        _tc_body,
        grid=(NBLK,),
        in_specs=[node_spec] * 6 + agg_specs + [
            pl.BlockSpec((HIDDEN, 1), lambda b: (0, 0)),
            pl.BlockSpec((HIDDEN, 1), lambda b: (0, 0)),
            pl.BlockSpec((HIDDEN, HIDDEN), lambda b: (0, 0)),
            pl.BlockSpec((HIDDEN, 1), lambda b: (0, 0)),
            pl.BlockSpec((OUT, HIDDEN), lambda b: (0, 0)),
            pl.BlockSpec((OUT, 1), lambda b: (0, 0)),
        ],
        out_specs=pl.BlockSpec((3, OUT, G), lambda b: (0, 0, 0)),
        out_shape=jax.ShapeDtypeStruct((3, OUT, G), jnp.float32),
        scratch_shapes=[
            pltpu.VMEM((HIDDEN, 3 * G), jnp.float32),
            pltpu.VMEM((G, 8), jnp.float32),
        ],
    )(xa, xp, xn, ba, bp, bn, agg, agg, agg, agg, agg, agg,
      W1.T, b1[:, None], W2.T, b2[:, None], fc_W.T, fc_b[:, None])


@jax.jit
def kernel(anchor_x, anchor_edge_index, anchor_batch,
           positive_x, positive_edge_index, positive_batch,
           negative_x, negative_edge_index, negative_batch,
           W1, b1, W2, b2, fc_W, fc_b):
    xa = anchor_x[:, 0]
    xp = positive_x[:, 0]
    xn = negative_x[:, 0]

    agg = _sc_call(xa, anchor_edge_index, xp, positive_edge_index,
                   xn, negative_edge_index)

    if False:
        d = agg[:G * OUT].reshape(G, OUT)
        return d, d, d
    outt = _tc_call(xa, xp, xn, anchor_batch, positive_batch, negative_batch,
                    agg, W1, b1, W2, b2, fc_W, fc_b)
    out = jnp.swapaxes(outt, 1, 2)
    return out[0], out[1], out[2]
